# Initial kernel scaffold; baseline (speedup 1.0000x reference)
import jax
import jax.numpy as jnp
from jax.experimental import pallas as pl


def kernel(feat_l, feat_r, edge_lr, edge_rl, edge_ll, edge_rr, w_lr, w_rl, w_ll, w_rr, W_inter, W_intra, W_ef_inter, b_ef_inter, W_ef_intra, b_ef_intra):
    raise NotImplementedError("write your pallas kernel here")


# speedup vs baseline: 2.1668x; 2.1668x over previous
import functools

import jax
import jax.numpy as jnp
from jax import lax
from jax.experimental import pallas as pl
from jax.experimental.pallas import tpu as pltpu
from jax.experimental.pallas import tpu_sc as plsc

N = 10000
NP = 10240
D = 128
E = 320000
NC = 2
NS = 16
K = 80
EPT = E // NS
NCHUNK = 256
EPTP = NCHUNK * K
PADE = EPTP - EPT
RPT = NP // NS
RC = 16
NRC = RPT // RC


def _tc_tables(x, w_inter, w_intra):
  nb = 10
  bm = (2 * N) // nb

  def body(x_ref, wi_ref, wt_ref, oi_ref, ot_ref):
    xv = x_ref[...]
    oi_ref[...] = jnp.dot(xv, wi_ref[...], preferred_element_type=jnp.float32)
    ot_ref[...] = jnp.dot(xv, wt_ref[...], preferred_element_type=jnp.float32)

  return pl.pallas_call(
      body,
      grid=(nb,),
      in_specs=[
          pl.BlockSpec((bm, D), lambda i: (i, 0)),
          pl.BlockSpec((D, D), lambda i: (0, 0)),
          pl.BlockSpec((D, D), lambda i: (0, 0)),
      ],
      out_specs=[
          pl.BlockSpec((bm, D), lambda i: (i, 0)),
          pl.BlockSpec((bm, D), lambda i: (i, 0)),
      ],
      out_shape=[
          jax.ShapeDtypeStruct((2 * N, D), jnp.float32),
          jax.ShapeDtypeStruct((2 * N, D), jnp.float32),
      ],
  )(x, w_inter, w_intra)


def _sc_scatter_mean(t_inter, t_intra, src1, dst1, src2, dst2):
  mesh = plsc.VectorSubcoreMesh(
      core_axis_name="c", subcore_axis_name="s",
      num_cores=NC, num_subcores=NS)

  @functools.partial(
      pl.kernel,
      out_type=(jax.ShapeDtypeStruct((2 * NP, D), jnp.float32),
                jax.ShapeDtypeStruct((2 * NP, D), jnp.float32)),
      mesh=mesh,
      scratch_types=[
          pltpu.VMEM((K,), jnp.int32),
          pltpu.VMEM((K,), jnp.int32),
          pltpu.VMEM((K, D), jnp.float32),
          pltpu.VMEM((RC, D), jnp.float32),
          pltpu.VMEM((RC, D), jnp.float32),
          pltpu.VMEM((RC, D), jnp.float32),
          pltpu.VMEM_SHARED((NP, D), jnp.float32),
          pltpu.SemaphoreType.DMA,
      ],
  )
  def k(ti_hbm, tt_hbm, s1_hbm, d1_hbm, s2_hbm, d2_hbm,
        out_hbm, rcp_hbm, sidx, didx, rows, obuf, pbuf, qbuf, acc, sem):
    c = lax.axis_index("c")
    s = lax.axis_index("s")
    t = c * NS + s
    ebase = t * EPTP
    rbase = s * RPT
    obase = c * NP + s * RPT

    def fill(value):
      vec = jnp.full((16,), value, jnp.float32)

      @pl.loop(0, K)
      def _(r):
        for cb in range(D // 16):
          rows[r, pl.ds(cb * 16, 16)] = vec

    def zero_acc():
      fill(0.0)

      @pl.loop(0, RPT // K)
      def _(z):
        pltpu.sync_copy(rows, acc.at[pl.ds(rbase + z * K, K)])

    def run_phase(table, s_hbm, d_hbm, first):
      zero_acc()
      fill(1.0)
      plsc.subcore_barrier()

      @pl.loop(0, NCHUNK)
      def _(i):
        pltpu.sync_copy(d_hbm.at[pl.ds(ebase + i * K, K)], didx)
        pltpu.sync_copy(rows, acc.at[didx], add=True)

      plsc.subcore_barrier()

      @pl.loop(0, NRC)
      def _(j):
        pltpu.sync_copy(acc.at[pl.ds(rbase + j * RC, RC)],
                        rows.at[pl.ds(0, RC)])

        @pl.loop(0, RC)
        def _(r):
          cv = rows[r, pl.ds(0, 16)]
          rcp = 1.0 / jnp.maximum(cv, 1.0)
          for cb in range(D // 16):
            obuf[r, pl.ds(cb * 16, 16)] = rcp

        pltpu.sync_copy(obuf, rcp_hbm.at[pl.ds(obase + j * RC, RC)])

      plsc.subcore_barrier()

      zero_acc()
      plsc.subcore_barrier()

      @pl.loop(0, NCHUNK)
      def _(i):
        pltpu.sync_copy(s_hbm.at[pl.ds(ebase + i * K, K)], sidx)
        pltpu.sync_copy(d_hbm.at[pl.ds(ebase + i * K, K)], didx)
        pltpu.async_copy(table.at[sidx], rows, sem).wait()
        pltpu.sync_copy(rows, acc.at[didx], add=True)

      plsc.subcore_barrier()

      @pl.loop(0, NRC)
      def _(j):
        pltpu.sync_copy(acc.at[pl.ds(rbase + j * RC, RC)],
                        rows.at[pl.ds(0, RC)])
        pltpu.sync_copy(rcp_hbm.at[pl.ds(obase + j * RC, RC)], pbuf)
        if not first:
          pltpu.sync_copy(out_hbm.at[pl.ds(obase + j * RC, RC)], qbuf)

        @pl.loop(0, RC)
        def _(r):
          for cb in range(D // 16):
            sl = pl.ds(cb * 16, 16)
            v = rows[r, sl] * pbuf[r, sl]
            if not first:
              v = jnp.maximum((v + qbuf[r, sl]) * 0.5, 0.0)
            obuf[r, sl] = v

        pltpu.sync_copy(obuf, out_hbm.at[pl.ds(obase + j * RC, RC)])

      plsc.subcore_barrier()

    run_phase(ti_hbm, s1_hbm, d1_hbm, True)
    run_phase(tt_hbm, s2_hbm, d2_hbm, False)

  return k(t_inter, t_intra, src1, dst1, src2, dst2)


def _pack_idx(a, pad_value):
  a = a.reshape(NS, EPT)
  pad = jnp.full((NS, PADE), pad_value, jnp.int32)
  return jnp.concatenate([a, pad], axis=1).reshape(-1)


def kernel(feat_l, feat_r, edge_lr, edge_rl, edge_ll, edge_rr,
           w_lr, w_rl, w_ll, w_rr,
           W_inter, W_intra, W_ef_inter, b_ef_inter, W_ef_intra, b_ef_intra):
  i32 = jnp.int32
  x = jnp.concatenate([feat_l, feat_r], axis=0)
  t_inter, t_intra = _tc_tables(x, W_inter, W_intra)

  src1 = jnp.concatenate([_pack_idx(edge_rl[0].astype(i32) + N, 0),
                          _pack_idx(edge_lr[0].astype(i32), 0)])
  dst1 = jnp.concatenate([_pack_idx(edge_rl[1].astype(i32), N),
                          _pack_idx(edge_lr[1].astype(i32), N)])
  src2 = jnp.concatenate([_pack_idx(edge_ll[0].astype(i32), 0),
                          _pack_idx(edge_rr[0].astype(i32) + N, 0)])
  dst2 = jnp.concatenate([_pack_idx(edge_ll[1].astype(i32), N),
                          _pack_idx(edge_rr[1].astype(i32), N)])

  out, _ = _sc_scatter_mean(t_inter, t_intra, src1, dst1, src2, dst2)
  return out[:N], out[NP:NP + N]


# speedup vs baseline: 2.7846x; 1.2852x over previous
import functools

import jax
import jax.numpy as jnp
from jax import lax
from jax.experimental import pallas as pl
from jax.experimental.pallas import tpu as pltpu
from jax.experimental.pallas import tpu_sc as plsc

N = 10000
NP = 10240
D = 128
E = 320000
NC = 2
NS = 16
K = 80
EPT = E // NS
NCHUNK = 256
EPTP = NCHUNK * K
PADE = EPTP - EPT
RPT = NP // NS
RC = 16
NRC = RPT // RC


def _tc_tables(x, w_inter, w_intra):
  nb = 10
  bm = (2 * N) // nb

  def body(x_ref, wi_ref, wt_ref, oi_ref, ot_ref):
    xv = x_ref[...]
    oi_ref[...] = jnp.dot(xv, wi_ref[...], preferred_element_type=jnp.float32)
    ot_ref[...] = jnp.dot(xv, wt_ref[...], preferred_element_type=jnp.float32)

  return pl.pallas_call(
      body,
      grid=(nb,),
      in_specs=[
          pl.BlockSpec((bm, D), lambda i: (i, 0)),
          pl.BlockSpec((D, D), lambda i: (0, 0)),
          pl.BlockSpec((D, D), lambda i: (0, 0)),
      ],
      out_specs=[
          pl.BlockSpec((bm, D), lambda i: (i, 0)),
          pl.BlockSpec((bm, D), lambda i: (i, 0)),
      ],
      out_shape=[
          jax.ShapeDtypeStruct((2 * N, D), jnp.float32),
          jax.ShapeDtypeStruct((2 * N, D), jnp.float32),
      ],
  )(x, w_inter, w_intra)


def _sc_scatter_mean(t_inter, t_intra, src1, dst1, src2, dst2):
  mesh = plsc.VectorSubcoreMesh(
      core_axis_name="c", subcore_axis_name="s",
      num_cores=NC, num_subcores=NS)

  @functools.partial(
      pl.kernel,
      out_type=(jax.ShapeDtypeStruct((2 * NP, D), jnp.float32),
                jax.ShapeDtypeStruct((2 * NP, D), jnp.float32)),
      mesh=mesh,
      scratch_types=[
          pltpu.VMEM((K,), jnp.int32),
          pltpu.VMEM((K,), jnp.int32),
          pltpu.VMEM((K,), jnp.int32),
          pltpu.VMEM((K,), jnp.int32),
          pltpu.VMEM((K, D), jnp.float32),
          pltpu.VMEM((K, D), jnp.float32),
          pltpu.VMEM((RC, D), jnp.float32),
          pltpu.VMEM((RC, D), jnp.float32),
          pltpu.VMEM((RC, D), jnp.float32),
          pltpu.VMEM_SHARED((NP, D), jnp.float32),
          pltpu.SemaphoreType.DMA,
          pltpu.SemaphoreType.DMA,
      ],
  )
  def k(ti_hbm, tt_hbm, s1_hbm, d1_hbm, s2_hbm, d2_hbm,
        out_hbm, rcp_hbm, sidx_a, sidx_b, didx_a, didx_b, rows, rows_b,
        obuf, pbuf, qbuf, acc, sem_a, sem_b):
    c = lax.axis_index("c")
    s = lax.axis_index("s")
    t = c * NS + s
    ebase = t * EPTP
    rbase = s * RPT
    obase = c * NP + s * RPT

    def fill(value):
      vec = jnp.full((16,), value, jnp.float32)

      @pl.loop(0, K)
      def _(r):
        for cb in range(D // 16):
          rows[r, pl.ds(cb * 16, 16)] = vec

    def zero_acc():
      fill(0.0)

      @pl.loop(0, RPT // K)
      def _(z):
        pltpu.sync_copy(rows, acc.at[pl.ds(rbase + z * K, K)])

    def run_phase(table, s_hbm, d_hbm, first):
      zero_acc()
      fill(1.0)
      plsc.subcore_barrier()

      pltpu.sync_copy(d_hbm.at[pl.ds(ebase, K)], didx_a)
      pltpu.async_copy(rows, acc.at[didx_a], sem_a, add=True)

      @pl.loop(0, NCHUNK // 2)
      def _(h):
        pltpu.sync_copy(d_hbm.at[pl.ds(ebase + (2 * h + 1) * K, K)], didx_b)
        pltpu.async_copy(rows, acc.at[didx_b], sem_b, add=True)
        pltpu.make_async_copy(rows, acc.at[didx_a], sem_a).wait()

        @pl.when(h + 1 < NCHUNK // 2)
        def _():
          pltpu.sync_copy(d_hbm.at[pl.ds(ebase + (2 * h + 2) * K, K)], didx_a)
          pltpu.async_copy(rows, acc.at[didx_a], sem_a, add=True)

        pltpu.make_async_copy(rows, acc.at[didx_b], sem_b).wait()

      plsc.subcore_barrier()

      @pl.loop(0, NRC)
      def _(j):
        pltpu.sync_copy(acc.at[pl.ds(rbase + j * RC, RC)],
                        rows.at[pl.ds(0, RC)])

        @pl.loop(0, RC)
        def _(r):
          cv = rows[r, pl.ds(0, 16)]
          rcp = 1.0 / jnp.maximum(cv, 1.0)
          for cb in range(D // 16):
            obuf[r, pl.ds(cb * 16, 16)] = rcp

        pltpu.sync_copy(obuf, rcp_hbm.at[pl.ds(obase + j * RC, RC)])

      plsc.subcore_barrier()

      zero_acc()
      plsc.subcore_barrier()

      pltpu.sync_copy(s_hbm.at[pl.ds(ebase, K)], sidx_a)
      pltpu.sync_copy(d_hbm.at[pl.ds(ebase, K)], didx_a)
      pltpu.async_copy(table.at[sidx_a], rows, sem_a)

      @pl.loop(0, NCHUNK // 2)
      def _(h):
        pltpu.sync_copy(s_hbm.at[pl.ds(ebase + (2 * h + 1) * K, K)], sidx_b)
        pltpu.sync_copy(d_hbm.at[pl.ds(ebase + (2 * h + 1) * K, K)], didx_b)
        pltpu.async_copy(table.at[sidx_b], rows_b, sem_b)
        pltpu.make_async_copy(table.at[sidx_a], rows, sem_a).wait()
        pltpu.sync_copy(rows, acc.at[didx_a], add=True)

        @pl.when(h + 1 < NCHUNK // 2)
        def _():
          pltpu.sync_copy(s_hbm.at[pl.ds(ebase + (2 * h + 2) * K, K)], sidx_a)
          pltpu.sync_copy(d_hbm.at[pl.ds(ebase + (2 * h + 2) * K, K)], didx_a)
          pltpu.async_copy(table.at[sidx_a], rows, sem_a)

        pltpu.make_async_copy(table.at[sidx_b], rows_b, sem_b).wait()
        pltpu.sync_copy(rows_b, acc.at[didx_b], add=True)

      plsc.subcore_barrier()

      @pl.loop(0, NRC)
      def _(j):
        pltpu.sync_copy(acc.at[pl.ds(rbase + j * RC, RC)],
                        rows.at[pl.ds(0, RC)])
        pltpu.sync_copy(rcp_hbm.at[pl.ds(obase + j * RC, RC)], pbuf)
        if not first:
          pltpu.sync_copy(out_hbm.at[pl.ds(obase + j * RC, RC)], qbuf)

        @pl.loop(0, RC)
        def _(r):
          for cb in range(D // 16):
            sl = pl.ds(cb * 16, 16)
            v = rows[r, sl] * pbuf[r, sl]
            if not first:
              v = jnp.maximum((v + qbuf[r, sl]) * 0.5, 0.0)
            obuf[r, sl] = v

        pltpu.sync_copy(obuf, out_hbm.at[pl.ds(obase + j * RC, RC)])

      plsc.subcore_barrier()

    run_phase(ti_hbm, s1_hbm, d1_hbm, True)
    run_phase(tt_hbm, s2_hbm, d2_hbm, False)

  return k(t_inter, t_intra, src1, dst1, src2, dst2)


def _pack_idx(a, pad_value):
  a = a.reshape(NS, EPT)
  pad = jnp.full((NS, PADE), pad_value, jnp.int32)
  return jnp.concatenate([a, pad], axis=1).reshape(-1)


def kernel(feat_l, feat_r, edge_lr, edge_rl, edge_ll, edge_rr,
           w_lr, w_rl, w_ll, w_rr,
           W_inter, W_intra, W_ef_inter, b_ef_inter, W_ef_intra, b_ef_intra):
  i32 = jnp.int32
  x = jnp.concatenate([feat_l, feat_r], axis=0)
  t_inter, t_intra = _tc_tables(x, W_inter, W_intra)

  src1 = jnp.concatenate([_pack_idx(edge_rl[0].astype(i32) + N, 0),
                          _pack_idx(edge_lr[0].astype(i32), 0)])
  dst1 = jnp.concatenate([_pack_idx(edge_rl[1].astype(i32), N),
                          _pack_idx(edge_lr[1].astype(i32), N)])
  src2 = jnp.concatenate([_pack_idx(edge_ll[0].astype(i32), 0),
                          _pack_idx(edge_rr[0].astype(i32) + N, 0)])
  dst2 = jnp.concatenate([_pack_idx(edge_ll[1].astype(i32), N),
                          _pack_idx(edge_rr[1].astype(i32), N)])

  out, _ = _sc_scatter_mean(t_inter, t_intra, src1, dst1, src2, dst2)
  return out[:N], out[NP:NP + N]


# speedup vs baseline: 2.8555x; 1.0254x over previous
import functools

import jax
import jax.numpy as jnp
from jax import lax
from jax.experimental import pallas as pl
from jax.experimental.pallas import tpu as pltpu
from jax.experimental.pallas import tpu_sc as plsc

N = 10000
NP = 10240
D = 128
E = 320000
NC = 2
NS = 16
K = 80
EPT = E // NS
NCHUNK = 256
EPTP = NCHUNK * K
PADE = EPTP - EPT
RPT = NP // NS
RC = 40
NRC = RPT // RC


def _tc_tables(x, w_inter, w_intra):
  nb = 10
  bm = (2 * N) // nb

  def body(x_ref, wi_ref, wt_ref, oi_ref, ot_ref):
    xv = x_ref[...]
    oi_ref[...] = jnp.dot(xv, wi_ref[...], preferred_element_type=jnp.float32)
    ot_ref[...] = jnp.dot(xv, wt_ref[...], preferred_element_type=jnp.float32)

  return pl.pallas_call(
      body,
      grid=(nb,),
      in_specs=[
          pl.BlockSpec((bm, D), lambda i: (i, 0)),
          pl.BlockSpec((D, D), lambda i: (0, 0)),
          pl.BlockSpec((D, D), lambda i: (0, 0)),
      ],
      out_specs=[
          pl.BlockSpec((bm, D), lambda i: (i, 0)),
          pl.BlockSpec((bm, D), lambda i: (i, 0)),
      ],
      out_shape=[
          jax.ShapeDtypeStruct((2 * N, D), jnp.float32),
          jax.ShapeDtypeStruct((2 * N, D), jnp.float32),
      ],
  )(x, w_inter, w_intra)


def _sc_scatter_mean(t_inter, t_intra, src1, dst1, src2, dst2):
  mesh = plsc.VectorSubcoreMesh(
      core_axis_name="c", subcore_axis_name="s",
      num_cores=NC, num_subcores=NS)

  @functools.partial(
      pl.kernel,
      out_type=(jax.ShapeDtypeStruct((2 * NP, D), jnp.float32),
                jax.ShapeDtypeStruct((2 * NP, D), jnp.float32)),
      mesh=mesh,
      scratch_types=[
          pltpu.VMEM((K,), jnp.int32),
          pltpu.VMEM((K,), jnp.int32),
          pltpu.VMEM((K,), jnp.int32),
          pltpu.VMEM((K,), jnp.int32),
          pltpu.VMEM((K, D), jnp.float32),
          pltpu.VMEM((K, D), jnp.float32),
          pltpu.VMEM_SHARED((NP, D), jnp.float32),
          pltpu.SemaphoreType.DMA,
          pltpu.SemaphoreType.DMA,
      ],
  )
  def k(ti_hbm, tt_hbm, s1_hbm, d1_hbm, s2_hbm, d2_hbm,
        out_hbm, rcp_hbm, sidx_a, sidx_b, didx_a, didx_b, rows, rows_b,
        acc, sem_a, sem_b):
    c = lax.axis_index("c")
    s = lax.axis_index("s")
    t = c * NS + s
    ebase = t * EPTP
    rbase = s * RPT
    obase = c * NP + s * RPT

    def fill(value):
      vec = jnp.full((16,), value, jnp.float32)

      @pl.loop(0, K)
      def _(r):
        for cb in range(D // 16):
          rows[r, pl.ds(cb * 16, 16)] = vec

    def zero_acc():
      fill(0.0)

      @pl.loop(0, RPT // K)
      def _(z):
        pltpu.sync_copy(rows, acc.at[pl.ds(rbase + z * K, K)])

    def run_phase(table, s_hbm, d_hbm, first):
      zero_acc()
      fill(1.0)
      plsc.subcore_barrier()

      pltpu.sync_copy(d_hbm.at[pl.ds(ebase, K)], didx_a)
      pltpu.async_copy(rows, acc.at[didx_a], sem_a, add=True)

      @pl.loop(0, NCHUNK // 2)
      def _(h):
        pltpu.sync_copy(d_hbm.at[pl.ds(ebase + (2 * h + 1) * K, K)], didx_b)
        pltpu.async_copy(rows, acc.at[didx_b], sem_b, add=True)
        pltpu.make_async_copy(rows, acc.at[didx_a], sem_a).wait()

        @pl.when(h + 1 < NCHUNK // 2)
        def _():
          pltpu.sync_copy(d_hbm.at[pl.ds(ebase + (2 * h + 2) * K, K)], didx_a)
          pltpu.async_copy(rows, acc.at[didx_a], sem_a, add=True)

        pltpu.make_async_copy(rows, acc.at[didx_b], sem_b).wait()

      plsc.subcore_barrier()

      @pl.loop(0, NRC)
      def _(j):
        pltpu.sync_copy(acc.at[pl.ds(rbase + j * RC, RC)],
                        rows.at[pl.ds(0, RC)])

        @pl.loop(0, RC)
        def _(r):
          cv = rows[r, pl.ds(0, 16)]
          rcp = 1.0 / jnp.maximum(cv, 1.0)
          for cb in range(D // 16):
            rows_b[r, pl.ds(cb * 16, 16)] = rcp

        pltpu.sync_copy(rows_b.at[pl.ds(0, RC)],
                        rcp_hbm.at[pl.ds(obase + j * RC, RC)])

      plsc.subcore_barrier()

      zero_acc()
      plsc.subcore_barrier()

      pltpu.sync_copy(s_hbm.at[pl.ds(ebase, K)], sidx_a)
      pltpu.sync_copy(d_hbm.at[pl.ds(ebase, K)], didx_a)
      pltpu.async_copy(table.at[sidx_a], rows, sem_a)

      @pl.loop(0, NCHUNK // 2)
      def _(h):
        pltpu.sync_copy(s_hbm.at[pl.ds(ebase + (2 * h + 1) * K, K)], sidx_b)
        pltpu.sync_copy(d_hbm.at[pl.ds(ebase + (2 * h + 1) * K, K)], didx_b)
        pltpu.async_copy(table.at[sidx_b], rows_b, sem_b)
        pltpu.make_async_copy(table.at[sidx_a], rows, sem_a).wait()
        pltpu.sync_copy(rows, acc.at[didx_a], add=True)

        @pl.when(h + 1 < NCHUNK // 2)
        def _():
          pltpu.sync_copy(s_hbm.at[pl.ds(ebase + (2 * h + 2) * K, K)], sidx_a)
          pltpu.sync_copy(d_hbm.at[pl.ds(ebase + (2 * h + 2) * K, K)], didx_a)
          pltpu.async_copy(table.at[sidx_a], rows, sem_a)

        pltpu.make_async_copy(table.at[sidx_b], rows_b, sem_b).wait()
        pltpu.sync_copy(rows_b, acc.at[didx_b], add=True)

      plsc.subcore_barrier()

      @pl.loop(0, NRC)
      def _(j):
        pltpu.sync_copy(acc.at[pl.ds(rbase + j * RC, RC)],
                        rows.at[pl.ds(0, RC)])
        pltpu.sync_copy(rcp_hbm.at[pl.ds(obase + j * RC, RC)],
                        rows_b.at[pl.ds(0, RC)])
        if not first:
          pltpu.sync_copy(out_hbm.at[pl.ds(obase + j * RC, RC)],
                          rows.at[pl.ds(RC, RC)])

        @pl.loop(0, RC)
        def _(r):
          for cb in range(D // 16):
            sl = pl.ds(cb * 16, 16)
            v = rows[r, sl] * rows_b[r, sl]
            if not first:
              v = jnp.maximum((v + rows[RC + r, sl]) * 0.5, 0.0)
            rows_b[RC + r, sl] = v

        pltpu.sync_copy(rows_b.at[pl.ds(RC, RC)],
                        out_hbm.at[pl.ds(obase + j * RC, RC)])

      plsc.subcore_barrier()

    run_phase(ti_hbm, s1_hbm, d1_hbm, True)
    run_phase(tt_hbm, s2_hbm, d2_hbm, False)

  return k(t_inter, t_intra, src1, dst1, src2, dst2)


def _pack_idx(a, pad_value):
  a = a.reshape(NS, EPT)
  pad = jnp.full((NS, PADE), pad_value, jnp.int32)
  return jnp.concatenate([a, pad], axis=1).reshape(-1)


def kernel(feat_l, feat_r, edge_lr, edge_rl, edge_ll, edge_rr,
           w_lr, w_rl, w_ll, w_rr,
           W_inter, W_intra, W_ef_inter, b_ef_inter, W_ef_intra, b_ef_intra):
  i32 = jnp.int32
  x = jnp.concatenate([feat_l, feat_r], axis=0)
  t_inter, t_intra = _tc_tables(x, W_inter, W_intra)

  src1 = jnp.concatenate([_pack_idx(edge_rl[0].astype(i32) + N, 0),
                          _pack_idx(edge_lr[0].astype(i32), 0)])
  dst1 = jnp.concatenate([_pack_idx(edge_rl[1].astype(i32), N),
                          _pack_idx(edge_lr[1].astype(i32), N)])
  src2 = jnp.concatenate([_pack_idx(edge_ll[0].astype(i32), 0),
                          _pack_idx(edge_rr[0].astype(i32) + N, 0)])
  dst2 = jnp.concatenate([_pack_idx(edge_ll[1].astype(i32), N),
                          _pack_idx(edge_rr[1].astype(i32), N)])

  out, _ = _sc_scatter_mean(t_inter, t_intra, src1, dst1, src2, dst2)
  return out[:N], out[NP:NP + N]


# speedup vs baseline: 3.0300x; 1.0611x over previous
import functools

import jax
import jax.numpy as jnp
from jax import lax
from jax.experimental import pallas as pl
from jax.experimental.pallas import tpu as pltpu
from jax.experimental.pallas import tpu_sc as plsc

N = 10000
NP = 10240
D = 128
E = 320000
NC = 2
NS = 16
K = 128
EPT = E // NS
NCHUNK = 160
EPTP = NCHUNK * K
PADE = EPTP - EPT
RPT = NP // NS
RC = 40
NRC = RPT // RC


def _tc_tables(x, w_inter, w_intra):
  nb = 10
  bm = (2 * N) // nb

  def body(x_ref, wi_ref, wt_ref, oi_ref, ot_ref):
    xv = x_ref[...]
    oi_ref[...] = jnp.dot(xv, wi_ref[...], preferred_element_type=jnp.float32)
    ot_ref[...] = jnp.dot(xv, wt_ref[...], preferred_element_type=jnp.float32)

  return pl.pallas_call(
      body,
      grid=(nb,),
      in_specs=[
          pl.BlockSpec((bm, D), lambda i: (i, 0)),
          pl.BlockSpec((D, D), lambda i: (0, 0)),
          pl.BlockSpec((D, D), lambda i: (0, 0)),
      ],
      out_specs=[
          pl.BlockSpec((bm, D), lambda i: (i, 0)),
          pl.BlockSpec((bm, D), lambda i: (i, 0)),
      ],
      out_shape=[
          jax.ShapeDtypeStruct((2 * N, D), jnp.float32),
          jax.ShapeDtypeStruct((2 * N, D), jnp.float32),
      ],
  )(x, w_inter, w_intra)


def _sc_scatter_mean(t_inter, t_intra, src1, dst1, src2, dst2):
  mesh = plsc.VectorSubcoreMesh(
      core_axis_name="c", subcore_axis_name="s",
      num_cores=NC, num_subcores=NS)

  @functools.partial(
      pl.kernel,
      out_type=(jax.ShapeDtypeStruct((2 * NP, D), jnp.float32),
                jax.ShapeDtypeStruct((2 * NP, D), jnp.float32)),
      mesh=mesh,
      scratch_types=[
          pltpu.VMEM((K,), jnp.int32),
          pltpu.VMEM((K,), jnp.int32),
          pltpu.VMEM((K,), jnp.int32),
          pltpu.VMEM((K,), jnp.int32),
          pltpu.VMEM((K, D), jnp.float32),
          pltpu.VMEM((K, D), jnp.float32),
          pltpu.VMEM_SHARED((NP, D), jnp.float32),
          pltpu.SemaphoreType.DMA,
          pltpu.SemaphoreType.DMA,
      ],
  )
  def k(ti_hbm, tt_hbm, s1_hbm, d1_hbm, s2_hbm, d2_hbm,
        out_hbm, rcp_hbm, sidx_a, sidx_b, didx_a, didx_b, rows, rows_b,
        acc, sem_a, sem_b):
    c = lax.axis_index("c")
    s = lax.axis_index("s")
    t = c * NS + s
    ebase = t * EPTP
    rbase = s * RPT
    obase = c * NP + s * RPT

    def fill(value):
      vec = jnp.full((16,), value, jnp.float32)

      @pl.loop(0, K)
      def _(r):
        for cb in range(D // 16):
          rows[r, pl.ds(cb * 16, 16)] = vec

    def zero_acc():
      fill(0.0)

      @pl.loop(0, RPT // K)
      def _(z):
        pltpu.sync_copy(rows, acc.at[pl.ds(rbase + z * K, K)])

    def run_phase(table, s_hbm, d_hbm, first):
      zero_acc()
      fill(1.0)
      plsc.subcore_barrier()

      pltpu.sync_copy(d_hbm.at[pl.ds(ebase, K)], didx_a)
      pltpu.async_copy(rows, acc.at[didx_a], sem_a, add=True)

      @pl.loop(0, NCHUNK // 2)
      def _(h):
        pltpu.sync_copy(d_hbm.at[pl.ds(ebase + (2 * h + 1) * K, K)], didx_b)
        pltpu.async_copy(rows, acc.at[didx_b], sem_b, add=True)
        pltpu.make_async_copy(rows, acc.at[didx_a], sem_a).wait()

        @pl.when(h + 1 < NCHUNK // 2)
        def _():
          pltpu.sync_copy(d_hbm.at[pl.ds(ebase + (2 * h + 2) * K, K)], didx_a)
          pltpu.async_copy(rows, acc.at[didx_a], sem_a, add=True)

        pltpu.make_async_copy(rows, acc.at[didx_b], sem_b).wait()

      plsc.subcore_barrier()

      @pl.loop(0, NRC)
      def _(j):
        pltpu.sync_copy(acc.at[pl.ds(rbase + j * RC, RC)],
                        rows.at[pl.ds(0, RC)])

        @pl.loop(0, RC)
        def _(r):
          cv = rows[r, pl.ds(0, 16)]
          rcp = 1.0 / jnp.maximum(cv, 1.0)
          for cb in range(D // 16):
            rows_b[r, pl.ds(cb * 16, 16)] = rcp

        pltpu.sync_copy(rows_b.at[pl.ds(0, RC)],
                        rcp_hbm.at[pl.ds(obase + j * RC, RC)])

      plsc.subcore_barrier()

      zero_acc()
      plsc.subcore_barrier()

      pltpu.sync_copy(s_hbm.at[pl.ds(ebase, K)], sidx_a)
      pltpu.sync_copy(d_hbm.at[pl.ds(ebase, K)], didx_a)
      pltpu.async_copy(table.at[sidx_a], rows, sem_a)

      @pl.loop(0, NCHUNK // 2)
      def _(h):
        pltpu.sync_copy(s_hbm.at[pl.ds(ebase + (2 * h + 1) * K, K)], sidx_b)
        pltpu.sync_copy(d_hbm.at[pl.ds(ebase + (2 * h + 1) * K, K)], didx_b)
        pltpu.async_copy(table.at[sidx_b], rows_b, sem_b)
        pltpu.make_async_copy(table.at[sidx_a], rows, sem_a).wait()
        pltpu.sync_copy(rows, acc.at[didx_a], add=True)

        @pl.when(h + 1 < NCHUNK // 2)
        def _():
          pltpu.sync_copy(s_hbm.at[pl.ds(ebase + (2 * h + 2) * K, K)], sidx_a)
          pltpu.sync_copy(d_hbm.at[pl.ds(ebase + (2 * h + 2) * K, K)], didx_a)
          pltpu.async_copy(table.at[sidx_a], rows, sem_a)

        pltpu.make_async_copy(table.at[sidx_b], rows_b, sem_b).wait()
        pltpu.sync_copy(rows_b, acc.at[didx_b], add=True)

      plsc.subcore_barrier()

      @pl.loop(0, NRC)
      def _(j):
        pltpu.sync_copy(acc.at[pl.ds(rbase + j * RC, RC)],
                        rows.at[pl.ds(0, RC)])
        pltpu.sync_copy(rcp_hbm.at[pl.ds(obase + j * RC, RC)],
                        rows_b.at[pl.ds(0, RC)])
        if not first:
          pltpu.sync_copy(out_hbm.at[pl.ds(obase + j * RC, RC)],
                          rows.at[pl.ds(RC, RC)])

        @pl.loop(0, RC)
        def _(r):
          for cb in range(D // 16):
            sl = pl.ds(cb * 16, 16)
            v = rows[r, sl] * rows_b[r, sl]
            if not first:
              v = jnp.maximum((v + rows[RC + r, sl]) * 0.5, 0.0)
            rows_b[RC + r, sl] = v

        pltpu.sync_copy(rows_b.at[pl.ds(RC, RC)],
                        out_hbm.at[pl.ds(obase + j * RC, RC)])

      plsc.subcore_barrier()

    run_phase(ti_hbm, s1_hbm, d1_hbm, True)
    run_phase(tt_hbm, s2_hbm, d2_hbm, False)

  return k(t_inter, t_intra, src1, dst1, src2, dst2)


def _pack_idx(a, pad_value):
  a = a.reshape(NS, EPT)
  pad = jnp.full((NS, PADE), pad_value, jnp.int32)
  return jnp.concatenate([a, pad], axis=1).reshape(-1)


def kernel(feat_l, feat_r, edge_lr, edge_rl, edge_ll, edge_rr,
           w_lr, w_rl, w_ll, w_rr,
           W_inter, W_intra, W_ef_inter, b_ef_inter, W_ef_intra, b_ef_intra):
  i32 = jnp.int32
  x = jnp.concatenate([feat_l, feat_r], axis=0)
  t_inter, t_intra = _tc_tables(x, W_inter, W_intra)

  src1 = jnp.concatenate([_pack_idx(edge_rl[0].astype(i32) + N, 0),
                          _pack_idx(edge_lr[0].astype(i32), 0)])
  dst1 = jnp.concatenate([_pack_idx(edge_rl[1].astype(i32), N),
                          _pack_idx(edge_lr[1].astype(i32), N)])
  src2 = jnp.concatenate([_pack_idx(edge_ll[0].astype(i32), 0),
                          _pack_idx(edge_rr[0].astype(i32) + N, 0)])
  dst2 = jnp.concatenate([_pack_idx(edge_ll[1].astype(i32), N),
                          _pack_idx(edge_rr[1].astype(i32), N)])

  out, _ = _sc_scatter_mean(t_inter, t_intra, src1, dst1, src2, dst2)
  return out[:N], out[NP:NP + N]


# speedup vs baseline: 3.1224x; 1.0305x over previous
import functools

import jax
import jax.numpy as jnp
from jax import lax
from jax.experimental import pallas as pl
from jax.experimental.pallas import tpu as pltpu
from jax.experimental.pallas import tpu_sc as plsc

N = 10000
NP = 10240
D = 128
E = 320000
NC = 2
NS = 16
K = 128
EPT = E // NS
NCHUNK = 160
EPTP = NCHUNK * K
PADE = EPTP - EPT
RPT = NP // NS
RC = 40
NRC = RPT // RC
GB = 8
NG = NCHUNK // GB


def _tc_tables(x, w_inter, w_intra):
  nb = 10
  bm = (2 * N) // nb

  def body(x_ref, wi_ref, wt_ref, oi_ref, ot_ref):
    xv = x_ref[...]
    oi_ref[...] = jnp.dot(xv, wi_ref[...], preferred_element_type=jnp.float32)
    ot_ref[...] = jnp.dot(xv, wt_ref[...], preferred_element_type=jnp.float32)

  return pl.pallas_call(
      body,
      grid=(nb,),
      in_specs=[
          pl.BlockSpec((bm, D), lambda i: (i, 0)),
          pl.BlockSpec((D, D), lambda i: (0, 0)),
          pl.BlockSpec((D, D), lambda i: (0, 0)),
      ],
      out_specs=[
          pl.BlockSpec((bm, D), lambda i: (i, 0)),
          pl.BlockSpec((bm, D), lambda i: (i, 0)),
      ],
      out_shape=[
          jax.ShapeDtypeStruct((2 * N, D), jnp.float32),
          jax.ShapeDtypeStruct((2 * N, D), jnp.float32),
      ],
  )(x, w_inter, w_intra)


def _sc_scatter_mean(t_inter, t_intra, src1, dst1, src2, dst2):
  mesh = plsc.VectorSubcoreMesh(
      core_axis_name="c", subcore_axis_name="s",
      num_cores=NC, num_subcores=NS)

  @functools.partial(
      pl.kernel,
      out_type=(jax.ShapeDtypeStruct((2 * NP, D), jnp.float32),
                jax.ShapeDtypeStruct((2 * NP, D), jnp.float32)),
      mesh=mesh,
      scratch_types=[
          pltpu.VMEM((GB * K,), jnp.int32),
          pltpu.VMEM((GB * K,), jnp.int32),
          pltpu.VMEM((K, D), jnp.float32),
          pltpu.VMEM((K, D), jnp.float32),
          pltpu.VMEM_SHARED((NP, D), jnp.float32),
          pltpu.SemaphoreType.DMA,
          pltpu.SemaphoreType.DMA,
      ],
  )
  def k(ti_hbm, tt_hbm, s1_hbm, d1_hbm, s2_hbm, d2_hbm,
        out_hbm, rcp_hbm, sbig, dbig, rows, rows_b,
        acc, sem_a, sem_b):
    c = lax.axis_index("c")
    s = lax.axis_index("s")
    t = c * NS + s
    ebase = t * EPTP
    rbase = s * RPT
    obase = c * NP + s * RPT

    def fill(value):
      vec = jnp.full((16,), value, jnp.float32)

      @pl.loop(0, K)
      def _(r):
        for cb in range(D // 16):
          rows[r, pl.ds(cb * 16, 16)] = vec

    def zero_acc():
      fill(0.0)

      @pl.loop(0, RPT // K)
      def _(z):
        pltpu.sync_copy(rows, acc.at[pl.ds(rbase + z * K, K)])

    def run_phase(table, s_hbm, d_hbm, first):
      zero_acc()
      fill(1.0)
      plsc.subcore_barrier()

      @pl.loop(0, NG)
      def _(g):
        pltpu.sync_copy(d_hbm.at[pl.ds(ebase + g * GB * K, GB * K)], dbig)
        for i in range(GB):
          pltpu.async_copy(rows, acc.at[dbig.at[pl.ds(i * K, K)]],
                           sem_a, add=True)
        for i in range(GB):
          pltpu.make_async_copy(rows, acc.at[dbig.at[pl.ds(i * K, K)]],
                                sem_a).wait()

      plsc.subcore_barrier()

      @pl.loop(0, NRC)
      def _(j):
        pltpu.sync_copy(acc.at[pl.ds(rbase + j * RC, RC)],
                        rows.at[pl.ds(0, RC)])

        @pl.loop(0, RC)
        def _(r):
          cv = rows[r, pl.ds(0, 16)]
          rcp = 1.0 / jnp.maximum(cv, 1.0)
          for cb in range(D // 16):
            rows_b[r, pl.ds(cb * 16, 16)] = rcp

        pltpu.sync_copy(rows_b.at[pl.ds(0, RC)],
                        rcp_hbm.at[pl.ds(obase + j * RC, RC)])

      plsc.subcore_barrier()

      zero_acc()
      plsc.subcore_barrier()

      @pl.loop(0, NG)
      def _(g):
        pltpu.sync_copy(s_hbm.at[pl.ds(ebase + g * GB * K, GB * K)], sbig)
        pltpu.sync_copy(d_hbm.at[pl.ds(ebase + g * GB * K, GB * K)], dbig)
        sl = [pl.ds(i * K, K) for i in range(GB)]
        pltpu.async_copy(table.at[sbig.at[sl[0]]], rows, sem_a)
        for p in range(GB // 2):
          pltpu.async_copy(table.at[sbig.at[sl[2 * p + 1]]], rows_b, sem_b)
          pltpu.make_async_copy(table.at[sbig.at[sl[2 * p]]], rows,
                                sem_a).wait()
          pltpu.sync_copy(rows, acc.at[dbig.at[sl[2 * p]]], add=True)
          if 2 * p + 2 < GB:
            pltpu.async_copy(table.at[sbig.at[sl[2 * p + 2]]], rows, sem_a)
          pltpu.make_async_copy(table.at[sbig.at[sl[2 * p + 1]]], rows_b,
                                sem_b).wait()
          pltpu.sync_copy(rows_b, acc.at[dbig.at[sl[2 * p + 1]]], add=True)

      plsc.subcore_barrier()

      @pl.loop(0, NRC)
      def _(j):
        pltpu.sync_copy(acc.at[pl.ds(rbase + j * RC, RC)],
                        rows.at[pl.ds(0, RC)])
        pltpu.sync_copy(rcp_hbm.at[pl.ds(obase + j * RC, RC)],
                        rows_b.at[pl.ds(0, RC)])
        if not first:
          pltpu.sync_copy(out_hbm.at[pl.ds(obase + j * RC, RC)],
                          rows.at[pl.ds(RC, RC)])

        @pl.loop(0, RC)
        def _(r):
          for cb in range(D // 16):
            sl = pl.ds(cb * 16, 16)
            v = rows[r, sl] * rows_b[r, sl]
            if not first:
              v = jnp.maximum((v + rows[RC + r, sl]) * 0.5, 0.0)
            rows_b[RC + r, sl] = v

        pltpu.sync_copy(rows_b.at[pl.ds(RC, RC)],
                        out_hbm.at[pl.ds(obase + j * RC, RC)])

      plsc.subcore_barrier()

    run_phase(ti_hbm, s1_hbm, d1_hbm, True)
    run_phase(tt_hbm, s2_hbm, d2_hbm, False)

  return k(t_inter, t_intra, src1, dst1, src2, dst2)


def _pack_idx(a, pad_value):
  a = a.reshape(NS, EPT)
  pad = jnp.full((NS, PADE), pad_value, jnp.int32)
  return jnp.concatenate([a, pad], axis=1).reshape(-1)


def kernel(feat_l, feat_r, edge_lr, edge_rl, edge_ll, edge_rr,
           w_lr, w_rl, w_ll, w_rr,
           W_inter, W_intra, W_ef_inter, b_ef_inter, W_ef_intra, b_ef_intra):
  i32 = jnp.int32
  x = jnp.concatenate([feat_l, feat_r], axis=0)
  t_inter, t_intra = _tc_tables(x, W_inter, W_intra)

  src1 = jnp.concatenate([_pack_idx(edge_rl[0].astype(i32) + N, 0),
                          _pack_idx(edge_lr[0].astype(i32), 0)])
  dst1 = jnp.concatenate([_pack_idx(edge_rl[1].astype(i32), N),
                          _pack_idx(edge_lr[1].astype(i32), N)])
  src2 = jnp.concatenate([_pack_idx(edge_ll[0].astype(i32), 0),
                          _pack_idx(edge_rr[0].astype(i32) + N, 0)])
  dst2 = jnp.concatenate([_pack_idx(edge_ll[1].astype(i32), N),
                          _pack_idx(edge_rr[1].astype(i32), N)])

  out, _ = _sc_scatter_mean(t_inter, t_intra, src1, dst1, src2, dst2)
  return out[:N], out[NP:NP + N]


# speedup vs baseline: 3.1232x; 1.0002x over previous
import functools

import jax
import jax.numpy as jnp
from jax import lax
from jax.experimental import pallas as pl
from jax.experimental.pallas import tpu as pltpu
from jax.experimental.pallas import tpu_sc as plsc

N = 10000
NP = 10240
D = 128
E = 320000
NC = 2
NS = 16
K = 128
EPT = E // NS
NCHUNK = 160
EPTP = NCHUNK * K
PADE = EPTP - EPT
RPT = NP // NS
RC = 40
NRC = RPT // RC
GB = 8
NG = NCHUNK // GB


def _tc_tables(x, w_inter, w_intra):
  nb = 10
  bm = (2 * N) // nb

  def body(x_ref, wi_ref, wt_ref, oi_ref, ot_ref):
    xv = x_ref[...]
    oi_ref[...] = jnp.dot(xv, wi_ref[...], preferred_element_type=jnp.float32)
    ot_ref[...] = jnp.dot(xv, wt_ref[...], preferred_element_type=jnp.float32)

  return pl.pallas_call(
      body,
      grid=(nb,),
      in_specs=[
          pl.BlockSpec((bm, D), lambda i: (i, 0)),
          pl.BlockSpec((D, D), lambda i: (0, 0)),
          pl.BlockSpec((D, D), lambda i: (0, 0)),
      ],
      out_specs=[
          pl.BlockSpec((bm, D), lambda i: (i, 0)),
          pl.BlockSpec((bm, D), lambda i: (i, 0)),
      ],
      out_shape=[
          jax.ShapeDtypeStruct((2 * N, D), jnp.float32),
          jax.ShapeDtypeStruct((2 * N, D), jnp.float32),
      ],
  )(x, w_inter, w_intra)


def _sc_scatter_mean(t_inter, t_intra, src1, dst1, src2, dst2):
  mesh = plsc.VectorSubcoreMesh(
      core_axis_name="c", subcore_axis_name="s",
      num_cores=NC, num_subcores=NS)

  @functools.partial(
      pl.kernel,
      out_type=(jax.ShapeDtypeStruct((2 * NP, D), jnp.float32),
                jax.ShapeDtypeStruct((2 * NP, D), jnp.float32)),
      mesh=mesh,
      scratch_types=[
          pltpu.VMEM((GB * K,), jnp.int32),
          pltpu.VMEM((GB * K,), jnp.int32),
          pltpu.VMEM((K, D), jnp.float32),
          pltpu.VMEM((K, D), jnp.float32),
          pltpu.VMEM_SHARED((NP, D), jnp.float32),
          pltpu.SemaphoreType.DMA,
          pltpu.SemaphoreType.DMA,
          pltpu.SemaphoreType.DMA,
          pltpu.SemaphoreType.DMA,
      ],
  )
  def k(ti_hbm, tt_hbm, s1_hbm, d1_hbm, s2_hbm, d2_hbm,
        out_hbm, rcp_hbm, sbig, dbig, rows, rows_b,
        acc, sem_a, sem_b, sem_c, sem_d):
    c = lax.axis_index("c")
    s = lax.axis_index("s")
    t = c * NS + s
    ebase = t * EPTP
    rbase = s * RPT
    obase = c * NP + s * RPT

    def fill(value):
      vec = jnp.full((16,), value, jnp.float32)

      @pl.loop(0, K)
      def _(r):
        for cb in range(D // 16):
          rows[r, pl.ds(cb * 16, 16)] = vec

    def zero_acc():
      fill(0.0)

      @pl.loop(0, RPT // K)
      def _(z):
        pltpu.sync_copy(rows, acc.at[pl.ds(rbase + z * K, K)])

    def run_phase(table, s_hbm, d_hbm, first):
      with jax.named_scope("zero1"):
        zero_acc()
        fill(1.0)
        plsc.subcore_barrier()

      with jax.named_scope("counts"):
        @pl.loop(0, NG)
        def _(g):
          pltpu.sync_copy(d_hbm.at[pl.ds(ebase + g * GB * K, GB * K)], dbig)
          for i in range(GB):
            pltpu.async_copy(rows, acc.at[dbig.at[pl.ds(i * K, K)]],
                             sem_a, add=True)
          for i in range(GB):
            pltpu.make_async_copy(rows, acc.at[dbig.at[pl.ds(i * K, K)]],
                                  sem_a).wait()

        plsc.subcore_barrier()

      with jax.named_scope("rcp_ep"):
        @pl.loop(0, NRC)
        def _(j):
          pltpu.sync_copy(acc.at[pl.ds(rbase + j * RC, RC)],
                          rows.at[pl.ds(0, RC)])

          @pl.loop(0, RC)
          def _(r):
            cv = rows[r, pl.ds(0, 16)]
            rcp = 1.0 / jnp.maximum(cv, 1.0)
            for cb in range(D // 16):
              rows_b[r, pl.ds(cb * 16, 16)] = rcp

          pltpu.sync_copy(rows_b.at[pl.ds(0, RC)],
                          rcp_hbm.at[pl.ds(obase + j * RC, RC)])

        plsc.subcore_barrier()

      with jax.named_scope("zero2"):
        zero_acc()
        plsc.subcore_barrier()

      @pl.loop(0, NG)
      def _(g):
        pltpu.sync_copy(s_hbm.at[pl.ds(ebase + g * GB * K, GB * K)], sbig)
        pltpu.sync_copy(d_hbm.at[pl.ds(ebase + g * GB * K, GB * K)], dbig)
        sl = [pl.ds(i * K, K) for i in range(GB)]
        pltpu.async_copy(table.at[sbig.at[sl[0]]], rows, sem_a)
        for p in range(GB // 2):
          pltpu.async_copy(table.at[sbig.at[sl[2 * p + 1]]], rows_b, sem_b)
          pltpu.make_async_copy(table.at[sbig.at[sl[2 * p]]], rows,
                                sem_a).wait()
          pltpu.sync_copy(rows, acc.at[dbig.at[sl[2 * p]]], add=True)
          if 2 * p + 2 < GB:
            pltpu.async_copy(table.at[sbig.at[sl[2 * p + 2]]], rows, sem_a)
          pltpu.make_async_copy(table.at[sbig.at[sl[2 * p + 1]]], rows_b,
                                sem_b).wait()
          pltpu.sync_copy(rows_b, acc.at[dbig.at[sl[2 * p + 1]]], add=True)

      plsc.subcore_barrier()

      @pl.loop(0, NRC)
      def _(j):
        pltpu.sync_copy(acc.at[pl.ds(rbase + j * RC, RC)],
                        rows.at[pl.ds(0, RC)])
        pltpu.sync_copy(rcp_hbm.at[pl.ds(obase + j * RC, RC)],
                        rows_b.at[pl.ds(0, RC)])
        if not first:
          pltpu.sync_copy(out_hbm.at[pl.ds(obase + j * RC, RC)],
                          rows.at[pl.ds(RC, RC)])

        @pl.loop(0, RC)
        def _(r):
          for cb in range(D // 16):
            sl = pl.ds(cb * 16, 16)
            v = rows[r, sl] * rows_b[r, sl]
            if not first:
              v = jnp.maximum((v + rows[RC + r, sl]) * 0.5, 0.0)
            rows_b[RC + r, sl] = v

        pltpu.sync_copy(rows_b.at[pl.ds(RC, RC)],
                        out_hbm.at[pl.ds(obase + j * RC, RC)])

      plsc.subcore_barrier()

    run_phase(ti_hbm, s1_hbm, d1_hbm, True)
    run_phase(tt_hbm, s2_hbm, d2_hbm, False)

  return k(t_inter, t_intra, src1, dst1, src2, dst2)


def _pack_idx(a, pad_value):
  a = a.reshape(NS, EPT)
  pad = jnp.full((NS, PADE), pad_value, jnp.int32)
  return jnp.concatenate([a, pad], axis=1).reshape(-1)


def kernel(feat_l, feat_r, edge_lr, edge_rl, edge_ll, edge_rr,
           w_lr, w_rl, w_ll, w_rr,
           W_inter, W_intra, W_ef_inter, b_ef_inter, W_ef_intra, b_ef_intra):
  i32 = jnp.int32
  x = jnp.concatenate([feat_l, feat_r], axis=0)
  t_inter, t_intra = _tc_tables(x, W_inter, W_intra)

  src1 = jnp.concatenate([_pack_idx(edge_rl[0].astype(i32) + N, 0),
                          _pack_idx(edge_lr[0].astype(i32), 0)])
  dst1 = jnp.concatenate([_pack_idx(edge_rl[1].astype(i32), N),
                          _pack_idx(edge_lr[1].astype(i32), N)])
  src2 = jnp.concatenate([_pack_idx(edge_ll[0].astype(i32), 0),
                          _pack_idx(edge_rr[0].astype(i32) + N, 0)])
  dst2 = jnp.concatenate([_pack_idx(edge_ll[1].astype(i32), N),
                          _pack_idx(edge_rr[1].astype(i32), N)])

  out, _ = _sc_scatter_mean(t_inter, t_intra, src1, dst1, src2, dst2)
  return out[:N], out[NP:NP + N]


# speedup vs baseline: 3.1961x; 1.0233x over previous
import functools

import jax
import jax.numpy as jnp
from jax import lax
from jax.experimental import pallas as pl
from jax.experimental.pallas import tpu as pltpu
from jax.experimental.pallas import tpu_sc as plsc

N = 10000
NP = 10240
D = 128
E = 320000
NC = 2
NS = 16
K = 128
EPT = E // NS
NCHUNK = 160
EPTP = NCHUNK * K
PADE = EPTP - EPT
RPT = NP // NS
RC = 40
NRC = RPT // RC
GB = 16
NG = NCHUNK // GB


def _tc_tables(x, w_inter, w_intra):
  nb = 10
  bm = (2 * N) // nb

  def body(x_ref, wi_ref, wt_ref, oi_ref, ot_ref):
    xv = x_ref[...]
    oi_ref[...] = jnp.dot(xv, wi_ref[...], preferred_element_type=jnp.float32)
    ot_ref[...] = jnp.dot(xv, wt_ref[...], preferred_element_type=jnp.float32)

  return pl.pallas_call(
      body,
      grid=(nb,),
      in_specs=[
          pl.BlockSpec((bm, D), lambda i: (i, 0)),
          pl.BlockSpec((D, D), lambda i: (0, 0)),
          pl.BlockSpec((D, D), lambda i: (0, 0)),
      ],
      out_specs=[
          pl.BlockSpec((bm, D), lambda i: (i, 0)),
          pl.BlockSpec((bm, D), lambda i: (i, 0)),
      ],
      out_shape=[
          jax.ShapeDtypeStruct((2 * N, D), jnp.float32),
          jax.ShapeDtypeStruct((2 * N, D), jnp.float32),
      ],
  )(x, w_inter, w_intra)


def _sc_scatter_mean(t_inter, t_intra, src1, dst1, src2, dst2):
  mesh = plsc.VectorSubcoreMesh(
      core_axis_name="c", subcore_axis_name="s",
      num_cores=NC, num_subcores=NS)

  @functools.partial(
      pl.kernel,
      out_type=(jax.ShapeDtypeStruct((2 * NP, D), jnp.float32),
                jax.ShapeDtypeStruct((2 * NP, D), jnp.float32)),
      mesh=mesh,
      scratch_types=[
          pltpu.VMEM((GB * K,), jnp.int32),
          pltpu.VMEM((GB * K,), jnp.int32),
          pltpu.VMEM((K, D), jnp.float32),
          pltpu.VMEM((K, D), jnp.float32),
          pltpu.VMEM_SHARED((NP, D), jnp.float32),
          pltpu.SemaphoreType.DMA,
          pltpu.SemaphoreType.DMA,
          pltpu.SemaphoreType.DMA,
          pltpu.SemaphoreType.DMA,
      ],
  )
  def k(ti_hbm, tt_hbm, s1_hbm, d1_hbm, s2_hbm, d2_hbm,
        out_hbm, rcp_hbm, sbig, dbig, rows, rows_b,
        acc, sem_a, sem_b, sem_c, sem_d):
    c = lax.axis_index("c")
    s = lax.axis_index("s")
    t = c * NS + s
    ebase = t * EPTP
    rbase = s * RPT
    obase = c * NP + s * RPT

    def fill(value):
      vec = jnp.full((16,), value, jnp.float32)

      @pl.loop(0, K)
      def _(r):
        for cb in range(D // 16):
          rows[r, pl.ds(cb * 16, 16)] = vec

    def zero_acc():
      fill(0.0)

      @pl.loop(0, RPT // K)
      def _(z):
        pltpu.sync_copy(rows, acc.at[pl.ds(rbase + z * K, K)])

    def run_phase(table, s_hbm, d_hbm, first):
      with jax.named_scope("zero1"):
        zero_acc()
        fill(1.0)
        plsc.subcore_barrier()

      with jax.named_scope("counts"):
        @pl.loop(0, NG)
        def _(g):
          pltpu.sync_copy(d_hbm.at[pl.ds(ebase + g * GB * K, GB * K)], dbig)
          for i in range(GB):
            pltpu.async_copy(rows, acc.at[dbig.at[pl.ds(i * K, K)]],
                             sem_a, add=True)
          for i in range(GB):
            pltpu.make_async_copy(rows, acc.at[dbig.at[pl.ds(i * K, K)]],
                                  sem_a).wait()

        plsc.subcore_barrier()

      with jax.named_scope("rcp_ep"):
        @pl.loop(0, NRC)
        def _(j):
          pltpu.sync_copy(acc.at[pl.ds(rbase + j * RC, RC)],
                          rows.at[pl.ds(0, RC)])

          @pl.loop(0, RC)
          def _(r):
            cv = rows[r, pl.ds(0, 16)]
            rcp = 1.0 / jnp.maximum(cv, 1.0)
            for cb in range(D // 16):
              rows_b[r, pl.ds(cb * 16, 16)] = rcp

          pltpu.sync_copy(rows_b.at[pl.ds(0, RC)],
                          rcp_hbm.at[pl.ds(obase + j * RC, RC)])

        plsc.subcore_barrier()

      with jax.named_scope("zero2"):
        zero_acc()
        plsc.subcore_barrier()

      @pl.loop(0, NG)
      def _(g):
        pltpu.sync_copy(s_hbm.at[pl.ds(ebase + g * GB * K, GB * K)], sbig)
        pltpu.sync_copy(d_hbm.at[pl.ds(ebase + g * GB * K, GB * K)], dbig)
        sl = [pl.ds(i * K, K) for i in range(GB)]
        pltpu.async_copy(table.at[sbig.at[sl[0]]], rows, sem_a)
        for p in range(GB // 2):
          pltpu.async_copy(table.at[sbig.at[sl[2 * p + 1]]], rows_b, sem_b)
          pltpu.make_async_copy(table.at[sbig.at[sl[2 * p]]], rows,
                                sem_a).wait()
          pltpu.sync_copy(rows, acc.at[dbig.at[sl[2 * p]]], add=True)
          if 2 * p + 2 < GB:
            pltpu.async_copy(table.at[sbig.at[sl[2 * p + 2]]], rows, sem_a)
          pltpu.make_async_copy(table.at[sbig.at[sl[2 * p + 1]]], rows_b,
                                sem_b).wait()
          pltpu.sync_copy(rows_b, acc.at[dbig.at[sl[2 * p + 1]]], add=True)

      plsc.subcore_barrier()

      @pl.loop(0, NRC)
      def _(j):
        pltpu.sync_copy(acc.at[pl.ds(rbase + j * RC, RC)],
                        rows.at[pl.ds(0, RC)])
        pltpu.sync_copy(rcp_hbm.at[pl.ds(obase + j * RC, RC)],
                        rows_b.at[pl.ds(0, RC)])
        if not first:
          pltpu.sync_copy(out_hbm.at[pl.ds(obase + j * RC, RC)],
                          rows.at[pl.ds(RC, RC)])

        @pl.loop(0, RC)
        def _(r):
          for cb in range(D // 16):
            sl = pl.ds(cb * 16, 16)
            v = rows[r, sl] * rows_b[r, sl]
            if not first:
              v = jnp.maximum((v + rows[RC + r, sl]) * 0.5, 0.0)
            rows_b[RC + r, sl] = v

        pltpu.sync_copy(rows_b.at[pl.ds(RC, RC)],
                        out_hbm.at[pl.ds(obase + j * RC, RC)])

      plsc.subcore_barrier()

    run_phase(ti_hbm, s1_hbm, d1_hbm, True)
    run_phase(tt_hbm, s2_hbm, d2_hbm, False)

  return k(t_inter, t_intra, src1, dst1, src2, dst2)


def _pack_idx(a, pad_value):
  a = a.reshape(NS, EPT)
  pad = jnp.full((NS, PADE), pad_value, jnp.int32)
  return jnp.concatenate([a, pad], axis=1).reshape(-1)


def kernel(feat_l, feat_r, edge_lr, edge_rl, edge_ll, edge_rr,
           w_lr, w_rl, w_ll, w_rr,
           W_inter, W_intra, W_ef_inter, b_ef_inter, W_ef_intra, b_ef_intra):
  i32 = jnp.int32
  x = jnp.concatenate([feat_l, feat_r], axis=0)
  t_inter, t_intra = _tc_tables(x, W_inter, W_intra)

  src1 = jnp.concatenate([_pack_idx(edge_rl[0].astype(i32) + N, 0),
                          _pack_idx(edge_lr[0].astype(i32), 0)])
  dst1 = jnp.concatenate([_pack_idx(edge_rl[1].astype(i32), N),
                          _pack_idx(edge_lr[1].astype(i32), N)])
  src2 = jnp.concatenate([_pack_idx(edge_ll[0].astype(i32), 0),
                          _pack_idx(edge_rr[0].astype(i32) + N, 0)])
  dst2 = jnp.concatenate([_pack_idx(edge_ll[1].astype(i32), N),
                          _pack_idx(edge_rr[1].astype(i32), N)])

  out, _ = _sc_scatter_mean(t_inter, t_intra, src1, dst1, src2, dst2)
  return out[:N], out[NP:NP + N]
